# baseline (speedup 1.0000x reference)
import functools

import jax
import jax.numpy as jnp
from jax.experimental import pallas as pl
from jax.experimental.pallas import tpu as pltpu

_CODE_NUM = 8
_D = 3
_IDX = 1


def _interp_kernel(c0_ref, c1_ref, c2_ref, code_ref, out_ref, *, cn, c, tp, sub):
    grid_i = jax.lax.broadcasted_iota(jnp.int32, (cn, sub), 0).astype(jnp.float32)

    def hat(cref, s):
        scaled = (cref[pl.ds(s * sub, sub)] + 1.0) * ((cn - 1) / 2.0)
        return jnp.maximum(0.0, 1.0 - jnp.abs(grid_i - scaled))

    for s in range(tp // sub):
        h0 = hat(c0_ref, s)
        h1 = hat(c1_ref, s)
        h2 = hat(c2_ref, s)

        w_low = (h1[:, None, :] * h0[None, :, :]).reshape(cn * cn, sub)

        a = jnp.dot(code_ref[...], w_low,
                    preferred_element_type=jnp.float32)

        acc = a[0:c, :] * h2[0:1, :]
        for i in range(1, cn):
            acc = acc + a[i * c:(i + 1) * c, :] * h2[i:i + 1, :]

        out_ref[0, :, pl.ds(s * sub, sub)] = acc


@jax.jit
def kernel(coords, shape_code):
    b, p, d = coords.shape
    c = shape_code.shape[0]
    cn = _CODE_NUM
    nblk = cn ** d

    npts = b * p
    tp = p
    sub = 8192
    if tp % sub != 0:
        sub = 1024 if tp % 1024 == 0 else tp

    code = jax.lax.slice_in_dim(shape_code, _IDX * nblk, (_IDX + 1) * nblk, axis=1)
    code_r = (code.astype(jnp.float32)
              .reshape(c, cn, cn * cn)
              .transpose(1, 0, 2)
              .reshape(cn * c, cn * cn))

    kernel_fn = functools.partial(_interp_kernel, cn=cn, c=c, tp=tp, sub=sub)

    planes = [coords[:, :, j].reshape(npts).astype(jnp.float32)
              for j in range(d)]

    out = pl.pallas_call(
        kernel_fn,
        out_shape=jax.ShapeDtypeStruct((b, c, p), jnp.float32),
        grid=(b,),
        in_specs=[
            pl.BlockSpec((tp,), lambda i: (i,)),
            pl.BlockSpec((tp,), lambda i: (i,)),
            pl.BlockSpec((tp,), lambda i: (i,)),
            pl.BlockSpec((cn * c, cn * cn), lambda i: (0, 0)),
        ],
        out_specs=pl.BlockSpec((1, c, tp), lambda i: (i, 0, 0)),
        compiler_params=pltpu.CompilerParams(
            dimension_semantics=("arbitrary",),
            vmem_limit_bytes=58 * 1024 * 1024,
        ),
    )(*planes, code_r)

    return out.transpose(0, 2, 1)
